# baseline (speedup 1.0000x reference)
import functools

import jax
import jax.numpy as jnp
from jax import lax
from jax.experimental import pallas as pl
from jax.experimental.pallas import tpu as pltpu
from jax.experimental.pallas import tpu_sc as plsc

N = 10000
E = 320000
D = 128
NC, NS = 2, 16
U = 128
EUPAD = 2560
ROWS_PER_TILE = 640
NPAD = NS * ROWS_PER_TILE
ZR = 16


def _segsum_kernel(edge_split: bool):
  upt = EUPAD // (NC * NS) if edge_split else EUPAD // NS
  ic = 16
  mesh = plsc.VectorSubcoreMesh(core_axis_name="c", subcore_axis_name="s",
                                num_cores=NC, num_subcores=NS)

  @functools.partial(
      pl.kernel,
      out_type=jax.ShapeDtypeStruct((NC, NPAD, D), jnp.float32),
      mesh=mesh,
      scratch_types=[
          pltpu.VMEM((ic, U), jnp.int32),
          pltpu.VMEM((ic, U), jnp.int32),
          pltpu.VMEM((U, D), jnp.float32),
          pltpu.VMEM((U, D), jnp.float32),
          pltpu.VMEM((ZR, D), jnp.float32),
          pltpu.VMEM_SHARED((NPAD, D), jnp.float32),
          pltpu.SemaphoreType.DMA,
          pltpu.SemaphoreType.DMA,
          pltpu.SemaphoreType.DMA,
          pltpu.SemaphoreType.DMA,
      ],
  )
  def k(table_hbm, src_hbm, dst_hbm, out_hbm, srcb, dstb, rowb0, rowb1, zbuf,
        acc, gsem0, gsem1, ssem0, ssem1):
    c = lax.axis_index("c")
    s = lax.axis_index("s")
    start = (s * NC + c) * upt if edge_split else s * upt

    zv = jnp.zeros((16,), jnp.float32)
    for r in range(ZR):
      for j in range(D // 16):
        zbuf[r, pl.ds(j * 16, 16)] = zv
    base = s * ROWS_PER_TILE
    for t in range(ROWS_PER_TILE // ZR):
      pltpu.sync_copy(zbuf, acc.at[pl.ds(base + t * ZR, ZR)])
    plsc.subcore_barrier()

    bufs = (rowb0, rowb1)
    gsems = (gsem0, gsem1)
    ssems = (ssem0, ssem1)

    def chunk(t, carry):
      pltpu.sync_copy(src_hbm.at[c, pl.ds(start + t * ic, ic)], srcb)
      pltpu.sync_copy(dst_hbm.at[pl.ds(start + t * ic, ic)], dstb)
      cps = [None] * ic
      scs = [None] * ic
      cps[0] = pltpu.async_copy(table_hbm.at[srcb.at[0]], bufs[0], gsems[0])
      for u in range(ic):
        if u + 1 < ic:
          if u >= 1:
            scs[u - 1].wait()
          cps[u + 1] = pltpu.async_copy(
              table_hbm.at[srcb.at[u + 1]], bufs[(u + 1) % 2],
              gsems[(u + 1) % 2])
        cps[u].wait()
        scs[u] = pltpu.async_copy(bufs[u % 2], acc.at[dstb.at[u]],
                                  ssems[u % 2], add=True)
      scs[ic - 2].wait()
      scs[ic - 1].wait()
      return carry

    lax.fori_loop(0, upt // ic, chunk, 0)
    plsc.subcore_barrier()

    pltpu.sync_copy(acc.at[pl.ds(base, ROWS_PER_TILE)],
                    out_hbm.at[c, pl.ds(base, ROWS_PER_TILE)])

  return k


@functools.cache
def _segsum(edge_split: bool):
  return _segsum_kernel(edge_split)


def _tc1_body(x_ref, p_ref, w_ref, b_ref, o_ref):
  h = x_ref[...] + p_ref[0] + p_ref[1]
  z = lax.dot_general(h, w_ref[...], (((1,), (1,)), ((), ())),
                      preferred_element_type=jnp.float32)
  r = jnp.maximum(z + b_ref[...], 0.0)
  o_ref[0] = r[:, :D]
  o_ref[1] = r[:, D:]


def _tc2_body(h1_ref, a_ref, w2_ref, b2_ref, w3_ref, b3_ref, w4_ref, b4_ref,
              w5_ref, b5_ref, o_ref):
  h2 = jnp.concatenate([h1_ref[0] + a_ref[0], h1_ref[1] + a_ref[1]], axis=1)
  z2 = jnp.maximum(
      lax.dot_general(h2, w2_ref[...], (((1,), (1,)), ((), ())),
                      preferred_element_type=jnp.float32) + b2_ref[...], 0.0)
  z3 = jnp.maximum(
      lax.dot_general(z2, w3_ref[...], (((1,), (1,)), ((), ())),
                      preferred_element_type=jnp.float32) + b3_ref[...], 0.0)
  z4 = jnp.maximum(
      lax.dot_general(z3, w4_ref[...], (((1,), (1,)), ((), ())),
                      preferred_element_type=jnp.float32) + b4_ref[...], 0.0)
  o_ref[...] = jnp.sum(z4 * w5_ref[...], axis=1, keepdims=True) + b5_ref[0, 0]


BN = 2000


def _tc1(x, part, W1, b1):
  return pl.pallas_call(
      _tc1_body,
      grid=(N // BN,),
      in_specs=[
          pl.BlockSpec((BN, D), lambda i: (i, 0)),
          pl.BlockSpec((NC, BN, D), lambda i: (0, i, 0)),
          pl.BlockSpec((2 * D, D), lambda i: (0, 0)),
          pl.BlockSpec((1, 2 * D), lambda i: (0, 0)),
      ],
      out_specs=pl.BlockSpec((NC, BN, D), lambda i: (0, i, 0)),
      out_shape=jax.ShapeDtypeStruct((NC, N, D), jnp.float32),
  )(x, part, W1, b1.reshape(1, -1))


def _tc2(h1, agg, W2, b2, W3, b3, W4, b4, W5, b5):
  return pl.pallas_call(
      _tc2_body,
      grid=(N // BN,),
      in_specs=[
          pl.BlockSpec((NC, BN, D), lambda i: (0, i, 0)),
          pl.BlockSpec((NC, BN, D), lambda i: (0, i, 0)),
          pl.BlockSpec((4 * D, 2 * D), lambda i: (0, 0)),
          pl.BlockSpec((1, 4 * D), lambda i: (0, 0)),
          pl.BlockSpec((D, 4 * D), lambda i: (0, 0)),
          pl.BlockSpec((1, D), lambda i: (0, 0)),
          pl.BlockSpec((D, D), lambda i: (0, 0)),
          pl.BlockSpec((1, D), lambda i: (0, 0)),
          pl.BlockSpec((1, D), lambda i: (0, 0)),
          pl.BlockSpec(memory_space=pltpu.SMEM),
      ],
      out_specs=pl.BlockSpec((BN, 1), lambda i: (i, 0)),
      out_shape=jax.ShapeDtypeStruct((N, 1), jnp.float32),
  )(h1, agg, W2, b2.reshape(1, -1), W3, b3.reshape(1, -1), W4,
    b4.reshape(1, -1), W5, b5.reshape(1, -1))


def kernel(x, edge_tensor, W1, b1, W2, b2, W3, b3, W4, b4, W5, b5):
  src = edge_tensor[0]
  dst = edge_tensor[1]
  pad = EUPAD * U - E
  src_p = jnp.concatenate([src, jnp.zeros((pad,), jnp.int32)])
  dst_p = jnp.concatenate(
      [dst, N + (jnp.arange(pad, dtype=jnp.int32) % (NPAD - N))])
  dst3 = dst_p.reshape(EUPAD, U)
  srcs1 = jnp.stack([src_p, src_p]).reshape(NC, EUPAD, U)
  srcs2 = jnp.stack([src_p, src_p + N]).reshape(NC, EUPAD, U)

  part1 = _segsum(True)(x, srcs1, dst3)
  h1 = _tc1(x, part1, W1, b1)
  agg2 = _segsum(False)(h1.reshape(NC * N, D), srcs2, dst3)
  return _tc2(h1, agg2, W2, b2, W3, b3, W4, b4, W5, b5)


# speedup vs baseline: 2.8367x; 2.8367x over previous
import functools

import jax
import jax.numpy as jnp
from jax import lax
from jax.experimental import pallas as pl
from jax.experimental.pallas import tpu as pltpu
from jax.experimental.pallas import tpu_sc as plsc

N = 10000
E = 320000
D = 128
NC, NS = 2, 16
U = 128
EUPAD = 2560
ROWS_PER_TILE = 640
NPAD = NS * ROWS_PER_TILE
ZR = 16


def _segsum_kernel(edge_split: bool):
  upt = EUPAD // (NC * NS) if edge_split else EUPAD // NS
  ic = 16
  mesh = plsc.VectorSubcoreMesh(core_axis_name="c", subcore_axis_name="s",
                                num_cores=NC, num_subcores=NS)

  @functools.partial(
      pl.kernel,
      out_type=jax.ShapeDtypeStruct((NC, NPAD, D), jnp.float32),
      mesh=mesh,
      scratch_types=[
          pltpu.VMEM((ic, U), jnp.int32),
          pltpu.VMEM((ic, U), jnp.int32),
          pltpu.VMEM((U, D), jnp.float32),
          pltpu.VMEM((U, D), jnp.float32),
          pltpu.VMEM((ZR, D), jnp.float32),
          pltpu.VMEM_SHARED((NPAD, D), jnp.float32),
          pltpu.SemaphoreType.DMA,
          pltpu.SemaphoreType.DMA,
          pltpu.SemaphoreType.DMA,
          pltpu.SemaphoreType.DMA,
      ],
  )
  def k(table_hbm, src_hbm, dst_hbm, out_hbm, srcb, dstb, rowb0, rowb1, zbuf,
        acc, gsem0, gsem1, ssem0, ssem1):
    c = lax.axis_index("c")
    s = lax.axis_index("s")
    start = (s * NC + c) * upt if edge_split else s * upt

    zv = jnp.zeros((16,), jnp.float32)
    for r in range(ZR):
      for j in range(D // 16):
        zbuf[r, pl.ds(j * 16, 16)] = zv
    base = s * ROWS_PER_TILE
    for t in range(ROWS_PER_TILE // ZR):
      pltpu.sync_copy(zbuf, acc.at[pl.ds(base + t * ZR, ZR)])
    plsc.subcore_barrier()

    bufs = (rowb0, rowb1)
    gsems = (gsem0, gsem1)
    ssems = (ssem0, ssem1)

    def chunk(t, carry):
      pltpu.sync_copy(src_hbm.at[c, pl.ds(start + t * ic, ic)], srcb)
      pltpu.sync_copy(dst_hbm.at[pl.ds(start + t * ic, ic)], dstb)
      cps = [None] * ic
      scs = [None] * ic
      cps[0] = pltpu.async_copy(table_hbm.at[srcb.at[0]], bufs[0], gsems[0])
      for u in range(ic):
        if u + 1 < ic:
          if u >= 1:
            scs[u - 1].wait()
          cps[u + 1] = pltpu.async_copy(
              table_hbm.at[srcb.at[u + 1]], bufs[(u + 1) % 2],
              gsems[(u + 1) % 2])
        cps[u].wait()
        scs[u] = pltpu.async_copy(bufs[u % 2], acc.at[dstb.at[u]],
                                  ssems[u % 2], add=True)
      scs[ic - 2].wait()
      scs[ic - 1].wait()
      return carry

    lax.fori_loop(0, upt // ic, chunk, 0)
    plsc.subcore_barrier()

    pltpu.sync_copy(acc.at[pl.ds(base, ROWS_PER_TILE)],
                    out_hbm.at[c, pl.ds(base, ROWS_PER_TILE)])

  return k


@functools.cache
def _segsum(edge_split: bool):
  return _segsum_kernel(edge_split)


def _tc1_body(x_ref, p_ref, w_ref, b_ref, o_ref):
  h = x_ref[...] + p_ref[0] + p_ref[1]
  z = lax.dot_general(h, w_ref[...], (((1,), (1,)), ((), ())),
                      preferred_element_type=jnp.float32)
  r = jnp.maximum(z + b_ref[...], 0.0)
  o_ref[0] = r[:, :D]
  o_ref[1] = r[:, D:]


def _tc2_body(h1_ref, a_ref, w2_ref, b2_ref, w3_ref, b3_ref, w4_ref, b4_ref,
              w5_ref, b5_ref, o_ref):
  h2 = jnp.concatenate([h1_ref[0] + a_ref[0], h1_ref[1] + a_ref[1]], axis=1)
  z2 = jnp.maximum(
      lax.dot_general(h2, w2_ref[...], (((1,), (1,)), ((), ())),
                      preferred_element_type=jnp.float32) + b2_ref[...], 0.0)
  z3 = jnp.maximum(
      lax.dot_general(z2, w3_ref[...], (((1,), (1,)), ((), ())),
                      preferred_element_type=jnp.float32) + b3_ref[...], 0.0)
  z4 = jnp.maximum(
      lax.dot_general(z3, w4_ref[...], (((1,), (1,)), ((), ())),
                      preferred_element_type=jnp.float32) + b4_ref[...], 0.0)
  o_ref[...] = jnp.sum(z4 * w5_ref[...], axis=1, keepdims=True) + b5_ref[0, 0]


BN = 2000


def _tc1(x, part, W1, b1):
  return pl.pallas_call(
      _tc1_body,
      grid=(N // BN,),
      in_specs=[
          pl.BlockSpec((BN, D), lambda i: (i, 0)),
          pl.BlockSpec((NC, BN, D), lambda i: (0, i, 0)),
          pl.BlockSpec((2 * D, D), lambda i: (0, 0)),
          pl.BlockSpec((1, 2 * D), lambda i: (0, 0)),
      ],
      out_specs=pl.BlockSpec((NC, BN, D), lambda i: (0, i, 0)),
      out_shape=jax.ShapeDtypeStruct((NC, N, D), jnp.float32),
  )(x, part, W1, b1.reshape(1, -1))


def _tc2(h1, agg, W2, b2, W3, b3, W4, b4, W5, b5):
  return pl.pallas_call(
      _tc2_body,
      grid=(N // BN,),
      in_specs=[
          pl.BlockSpec((NC, BN, D), lambda i: (0, i, 0)),
          pl.BlockSpec((NC, BN, D), lambda i: (0, i, 0)),
          pl.BlockSpec((4 * D, 2 * D), lambda i: (0, 0)),
          pl.BlockSpec((1, 4 * D), lambda i: (0, 0)),
          pl.BlockSpec((D, 4 * D), lambda i: (0, 0)),
          pl.BlockSpec((1, D), lambda i: (0, 0)),
          pl.BlockSpec((D, D), lambda i: (0, 0)),
          pl.BlockSpec((1, D), lambda i: (0, 0)),
          pl.BlockSpec((1, D), lambda i: (0, 0)),
          pl.BlockSpec(memory_space=pltpu.SMEM),
      ],
      out_specs=pl.BlockSpec((BN, 1), lambda i: (i, 0)),
      out_shape=jax.ShapeDtypeStruct((N, 1), jnp.float32),
  )(h1, agg, W2, b2.reshape(1, -1), W3, b3.reshape(1, -1), W4,
    b4.reshape(1, -1), W5, b5.reshape(1, -1))


def kernel(x, edge_tensor, W1, b1, W2, b2, W3, b3, W4, b4, W5, b5):
  src = edge_tensor[0]
  dst = edge_tensor[1]
  pad = EUPAD * U - E
  src_p = jnp.concatenate([src, jnp.arange(pad, dtype=jnp.int32) % N])
  dst_p = jnp.concatenate(
      [dst, N + (jnp.arange(pad, dtype=jnp.int32) % (NPAD - N))])
  dst3 = dst_p.reshape(EUPAD, U)
  srcs1 = jnp.stack([src_p, src_p]).reshape(NC, EUPAD, U)
  srcs2 = jnp.stack([src_p, src_p + N]).reshape(NC, EUPAD, U)

  part1 = _segsum(True)(x, srcs1, dst3)
  h1 = _tc1(x, part1, W1, b1)
  agg2 = _segsum(False)(h1.reshape(NC * N, D), srcs2, dst3)
  return _tc2(h1, agg2, W2, b2, W3, b3, W4, b4, W5, b5)
